# baseline (speedup 1.0000x reference)
import jax
import jax.numpy as jnp
from jax.experimental import pallas as pl
from jax.experimental.pallas import tpu as pltpu

_MSPLIT = 2
_BM = 200


def _support_body(x_ref, w_ref, out_ref):
    out_ref[...] = jnp.dot(x_ref[...], w_ref[...],
                           preferred_element_type=jnp.float32).astype(jnp.bfloat16)


def _spmm_body(*refs):
    adj_refs = refs[:_MSPLIT]
    s_ref = refs[_MSPLIT]
    out_ref = refs[_MSPLIT + 1]
    for j in range(_MSPLIT):
        out_ref[j * _BM:(j + 1) * _BM, :] = jnp.dot(
            adj_refs[j][...].astype(jnp.bfloat16), s_ref[...],
            preferred_element_type=jnp.float32)


def kernel(x, adj, weight):
    n, in_c = x.shape
    out_c = weight.shape[1]

    support = pl.pallas_call(
        _support_body,
        out_shape=jax.ShapeDtypeStruct((n, out_c), jnp.bfloat16),
    )(x, weight)

    adj_specs = [
        pl.BlockSpec((_BM, n), lambda i, j=j: (i * _MSPLIT + j, 0))
        for j in range(_MSPLIT)
    ]
    out = pl.pallas_call(
        _spmm_body,
        grid=(n // (_BM * _MSPLIT),),
        in_specs=adj_specs + [pl.BlockSpec((n, out_c), lambda i: (0, 0))],
        out_specs=pl.BlockSpec((_BM * _MSPLIT, out_c), lambda i: (i, 0)),
        out_shape=jax.ShapeDtypeStruct((n, out_c), jnp.float32),
        compiler_params=pltpu.CompilerParams(
            dimension_semantics=("parallel",)),
    )(*([adj] * _MSPLIT), support)
    return out


# speedup vs baseline: 1.0333x; 1.0333x over previous
import jax
import jax.numpy as jnp
from jax.experimental import pallas as pl
from jax.experimental.pallas import tpu as pltpu

_MSPLIT = 2
_BM = 200


def _fused_body(*refs):
    adj_refs = refs[:_MSPLIT]
    x_ref, w_ref, out_ref, s_ref = refs[_MSPLIT:]

    @pl.when(pl.program_id(0) == 0)
    def _():
        s_ref[...] = jnp.dot(x_ref[...], w_ref[...],
                             preferred_element_type=jnp.float32).astype(jnp.bfloat16)

    for j in range(_MSPLIT):
        out_ref[j * _BM:(j + 1) * _BM, :] = jnp.dot(
            adj_refs[j][...].astype(jnp.bfloat16), s_ref[...],
            preferred_element_type=jnp.float32)


def kernel(x, adj, weight):
    n, in_c = x.shape
    out_c = weight.shape[1]

    adj_specs = [
        pl.BlockSpec((_BM, n), lambda i, j=j: (i * _MSPLIT + j, 0))
        for j in range(_MSPLIT)
    ]
    out = pl.pallas_call(
        _fused_body,
        grid=(n // (_BM * _MSPLIT),),
        in_specs=adj_specs + [
            pl.BlockSpec((n, in_c), lambda i: (0, 0)),
            pl.BlockSpec((in_c, out_c), lambda i: (0, 0)),
        ],
        out_specs=pl.BlockSpec((_BM * _MSPLIT, out_c), lambda i: (i, 0)),
        out_shape=jax.ShapeDtypeStruct((n, out_c), jnp.float32),
        scratch_shapes=[pltpu.VMEM((n, out_c), jnp.bfloat16)],
        compiler_params=pltpu.CompilerParams(
            dimension_semantics=("arbitrary",)),
    )(*([adj] * _MSPLIT), x, weight)
    return out


# speedup vs baseline: 1.0376x; 1.0042x over previous
import jax
import jax.numpy as jnp
from jax.experimental import pallas as pl
from jax.experimental.pallas import tpu as pltpu

_MSPLIT = 1
_BM = 400


def _fused_body(*refs):
    adj_refs = refs[:_MSPLIT]
    x_ref, w_ref, out_ref, s_ref = refs[_MSPLIT:]

    @pl.when(pl.program_id(0) == 0)
    def _():
        s_ref[...] = jnp.dot(x_ref[...], w_ref[...],
                             preferred_element_type=jnp.float32).astype(jnp.bfloat16)

    for j in range(_MSPLIT):
        out_ref[j * _BM:(j + 1) * _BM, :] = jnp.dot(
            adj_refs[j][...].astype(jnp.bfloat16), s_ref[...],
            preferred_element_type=jnp.float32)


def kernel(x, adj, weight):
    n, in_c = x.shape
    out_c = weight.shape[1]

    adj_specs = [
        pl.BlockSpec((_BM, n), lambda i, j=j: (i * _MSPLIT + j, 0))
        for j in range(_MSPLIT)
    ]
    out = pl.pallas_call(
        _fused_body,
        grid=(n // (_BM * _MSPLIT),),
        in_specs=adj_specs + [
            pl.BlockSpec((n, in_c), lambda i: (0, 0)),
            pl.BlockSpec((in_c, out_c), lambda i: (0, 0)),
        ],
        out_specs=pl.BlockSpec((_BM * _MSPLIT, out_c), lambda i: (i, 0)),
        out_shape=jax.ShapeDtypeStruct((n, out_c), jnp.float32),
        scratch_shapes=[pltpu.VMEM((n, out_c), jnp.bfloat16)],
        compiler_params=pltpu.CompilerParams(
            dimension_semantics=("arbitrary",)),
    )(*([adj] * _MSPLIT), x, weight)
    return out
